# baseline (speedup 1.0000x reference)
import functools

import jax
import jax.numpy as jnp
from jax.experimental import pallas as pl
from jax.experimental.pallas import tpu as pltpu

B, N1, N2, F = 10000, 8, 8, 128
AGG, OUT, LBL = 128, 128, 50
BB = 400
H = N1 // 2
CH = 1600


def _l2norm(x):
    s = jnp.sum(x * x, axis=-1, keepdims=True)
    return x * jax.lax.rsqrt(jnp.maximum(s, 1e-12))


def _fused_kernel(hop2a_ref, hop2b_ref, hop1_ref, target_ref,
                  wagg0_ref, wagg1_ref,
                  wc0x_ref, wc0a_ref, wc1t_ref, wc1a_ref,
                  wcls_ref, out_ref):
    dot = functools.partial(jnp.dot, preferred_element_type=jnp.float32)
    wagg0 = wagg0_ref[...]
    wagg1 = wagg1_ref[...]
    wc0x = wc0x_ref[...]
    wc0a = wc0a_ref[...]

    hop1_all = hop1_ref[...]

    def half(x2_ref, hop1_half):
        x2 = x2_ref[...].reshape(BB * H * N2, F)
        chunks = []
        for c in range(0, BB * H * N2, CH):
            pc = jax.nn.relu(dot(x2[c:c + CH], wagg0))
            chunks.append(jnp.sum(pc.reshape(CH // N2, N2, AGG), axis=1))
        a_h2 = jnp.concatenate(chunks, axis=0)

        hop1 = hop1_half.reshape(BB * H, F)
        h1 = _l2norm(jax.nn.relu(dot(hop1, wc0x) + dot(a_h2, wc0a)))

        s_h1 = jnp.sum(jax.nn.relu(dot(hop1, wagg0)).reshape(BB, H, AGG), axis=1)
        s_l1 = jnp.sum(jax.nn.relu(dot(h1, wagg1)).reshape(BB, H, AGG), axis=1)
        return s_h1, s_l1

    s_h1a, s_l1a = half(hop2a_ref, hop1_all[:, :H, :])
    s_h1b, s_l1b = half(hop2b_ref, hop1_all[:, H:, :])
    a_h1 = s_h1a + s_h1b
    a_l1 = s_l1a + s_l1b

    t = _l2norm(jax.nn.relu(dot(target_ref[...], wc0x) + dot(a_h1, wc0a)))

    full = _l2norm(dot(t, wc1t_ref[...]) + dot(a_l1, wc1a_ref[...]))
    full = _l2norm(full)

    out_ref[...] = jax.nn.relu(dot(full, wcls_ref[...]))


def kernel(hop2, hop1, target, W_agg0, W_agg1, W_comb0, W_comb1, W_cls):
    wagg0 = W_agg0 * (1.0 / N2)
    wagg1 = W_agg1 * (1.0 / N1)
    wc0x, wc0a = W_comb0[:F], W_comb0[F:]
    wc1t, wc1a = W_comb1[:OUT], W_comb1[OUT:]

    h2f = hop2.reshape(B, N1 * N2, F)
    grid = (B // BB,)
    full_w = lambda shape: pl.BlockSpec(shape, lambda i: (0,) * len(shape))
    out = pl.pallas_call(
        _fused_kernel,
        grid=grid,
        in_specs=[
            pl.BlockSpec((BB, H * N2, F), lambda i: (i, 0, 0)),
            pl.BlockSpec((BB, H * N2, F), lambda i: (i, 1, 0)),
            pl.BlockSpec((BB, N1, F), lambda i: (i, 0, 0)),
            pl.BlockSpec((BB, F), lambda i: (i, 0)),
            full_w((F, AGG)),
            full_w((OUT, AGG)),
            full_w((F, OUT)),
            full_w((AGG, OUT)),
            full_w((OUT, OUT)),
            full_w((AGG, OUT)),
            full_w((OUT, LBL)),
        ],
        out_specs=pl.BlockSpec((BB, LBL), lambda i: (i, 0)),
        out_shape=jax.ShapeDtypeStruct((B, LBL), jnp.float32),
        compiler_params=pltpu.CompilerParams(
            dimension_semantics=("arbitrary",),
        ),
    )(h2f, h2f, hop1, target,
      wagg0, wagg1, wc0x, wc0a, wc1t, wc1a, W_cls)
    return out


# speedup vs baseline: 1.1138x; 1.1138x over previous
import functools

import jax
import jax.numpy as jnp
from jax.experimental import pallas as pl
from jax.experimental.pallas import tpu as pltpu

B, N1, N2, F = 10000, 8, 8, 128
AGG, OUT, LBL = 128, 128, 50
HB = 200
CH = 1600


def _l2norm(x):
    s = jnp.sum(x * x, axis=-1, keepdims=True)
    return x * jax.lax.rsqrt(jnp.maximum(s, 1e-12))


def _fused_kernel(hop2a_ref, hop2b_ref, hop1_ref, target_ref,
                  wagg0_ref, wagg1_ref,
                  wc0x_ref, wc0a_ref, wc1t_ref, wc1a_ref,
                  wcls_ref, out_ref):
    dot = functools.partial(jnp.dot, preferred_element_type=jnp.float32)
    wagg0 = wagg0_ref[...]
    wagg1 = wagg1_ref[...]
    wc0x = wc0x_ref[...]
    wc0a = wc0a_ref[...]

    def pipe(x2_ref, hop1, target):
        x2 = x2_ref[...].reshape(HB * N1 * N2, F)
        chunks = []
        for c in range(0, HB * N1 * N2, CH):
            pc = jax.nn.relu(dot(x2[c:c + CH], wagg0))
            chunks.append(jnp.sum(pc.reshape(CH // N2, N2, AGG), axis=1))
        a_h2 = jnp.concatenate(chunks, axis=0)

        hop1f = hop1.reshape(HB * N1, F)
        h1 = _l2norm(jax.nn.relu(dot(hop1f, wc0x) + dot(a_h2, wc0a)))

        a_h1 = jnp.sum(jax.nn.relu(dot(hop1f, wagg0)).reshape(HB, N1, AGG), axis=1)

        t = _l2norm(jax.nn.relu(dot(target, wc0x) + dot(a_h1, wc0a)))

        a_l1 = jnp.sum(jax.nn.relu(dot(h1, wagg1)).reshape(HB, N1, AGG), axis=1)

        full = _l2norm(dot(t, wc1t_ref[...]) + dot(a_l1, wc1a_ref[...]))
        full = _l2norm(full)

        return jax.nn.relu(dot(full, wcls_ref[...]))

    hop1 = hop1_ref[...]
    target = target_ref[...]
    out_ref[:HB, :] = pipe(hop2a_ref, hop1[:HB], target[:HB])
    out_ref[HB:, :] = pipe(hop2b_ref, hop1[HB:], target[HB:])


def kernel(hop2, hop1, target, W_agg0, W_agg1, W_comb0, W_comb1, W_cls):
    wagg0 = W_agg0 * (1.0 / N2)
    wagg1 = W_agg1 * (1.0 / N1)
    wc0x, wc0a = W_comb0[:F], W_comb0[F:]
    wc1t, wc1a = W_comb1[:OUT], W_comb1[OUT:]

    h2f = hop2.reshape(B, N1 * N2, F)
    grid = (B // (2 * HB),)
    full_w = lambda shape: pl.BlockSpec(shape, lambda i: (0,) * len(shape))
    out = pl.pallas_call(
        _fused_kernel,
        grid=grid,
        in_specs=[
            pl.BlockSpec((HB, N1 * N2, F), lambda i: (2 * i, 0, 0)),
            pl.BlockSpec((HB, N1 * N2, F), lambda i: (2 * i + 1, 0, 0)),
            pl.BlockSpec((2 * HB, N1, F), lambda i: (i, 0, 0)),
            pl.BlockSpec((2 * HB, F), lambda i: (i, 0)),
            full_w((F, AGG)),
            full_w((OUT, AGG)),
            full_w((F, OUT)),
            full_w((AGG, OUT)),
            full_w((OUT, OUT)),
            full_w((AGG, OUT)),
            full_w((OUT, LBL)),
        ],
        out_specs=pl.BlockSpec((2 * HB, LBL), lambda i: (i, 0)),
        out_shape=jax.ShapeDtypeStruct((B, LBL), jnp.float32),
        compiler_params=pltpu.CompilerParams(
            dimension_semantics=("arbitrary",),
        ),
    )(h2f, h2f, hop1, target,
      wagg0, wagg1, wc0x, wc0a, wc1t, wc1a, W_cls)
    return out


# speedup vs baseline: 1.1922x; 1.0704x over previous
import functools

import jax
import jax.numpy as jnp
from jax.experimental import pallas as pl
from jax.experimental.pallas import tpu as pltpu

B, N1, N2, F = 10000, 8, 8, 128
AGG, OUT, LBL = 128, 128, 50
BB = 400


def _l2norm(x):
    s = jnp.sum(x * x, axis=-1, keepdims=True)
    return x * jax.lax.rsqrt(jnp.maximum(s, 1e-12))


def _fused_kernel(hop2_ref, hop1_ref, target_ref,
                  wagg0_ref, wagg1_ref,
                  wc0x_ref, wc0a_ref, wc1t_ref, wc1a_ref,
                  wcls_ref, out_ref):
    dot = functools.partial(jnp.dot, preferred_element_type=jnp.float32)
    wagg0 = wagg0_ref[...]

    x2 = hop2_ref[...].reshape(BB * N1 * N2, F)
    CH = 1024
    chunks = []
    for c in range(0, BB * N1 * N2, CH):
        pc = jax.nn.relu(dot(x2[c:c + CH], wagg0))
        pr = jnp.transpose(pc.reshape(CH // N2, N2, AGG), (1, 0, 2))
        chunks.append(jnp.sum(pr, axis=0))
    a_h2 = jnp.concatenate(chunks, axis=0)

    hop1 = hop1_ref[...].reshape(BB * N1, F)
    h1 = _l2norm(jax.nn.relu(dot(hop1, wc0x_ref[...]) + dot(a_h2, wc0a_ref[...])))

    a_h1 = jnp.sum(jax.nn.relu(dot(hop1, wagg0)).reshape(BB, N1, AGG), axis=1)

    t = _l2norm(jax.nn.relu(dot(target_ref[...], wc0x_ref[...]) + dot(a_h1, wc0a_ref[...])))

    a_l1 = jnp.sum(jax.nn.relu(dot(h1, wagg1_ref[...])).reshape(BB, N1, AGG), axis=1)

    full = _l2norm(dot(t, wc1t_ref[...]) + dot(a_l1, wc1a_ref[...]))
    full = _l2norm(full)

    out_ref[...] = jax.nn.relu(dot(full, wcls_ref[...]))


def kernel(hop2, hop1, target, W_agg0, W_agg1, W_comb0, W_comb1, W_cls):
    wagg0 = W_agg0 * (1.0 / N2)
    wagg1 = W_agg1 * (1.0 / N1)
    wc0x, wc0a = W_comb0[:F], W_comb0[F:]
    wc1t, wc1a = W_comb1[:OUT], W_comb1[OUT:]

    grid = (B // BB,)
    full_w = lambda shape: pl.BlockSpec(shape, lambda i: (0,) * len(shape))
    out = pl.pallas_call(
        _fused_kernel,
        grid=grid,
        in_specs=[
            pl.BlockSpec((BB, N1 * N2, F), lambda i: (i, 0, 0)),
            pl.BlockSpec((BB, N1, F), lambda i: (i, 0, 0)),
            pl.BlockSpec((BB, F), lambda i: (i, 0)),
            full_w((F, AGG)),
            full_w((OUT, AGG)),
            full_w((F, OUT)),
            full_w((AGG, OUT)),
            full_w((OUT, OUT)),
            full_w((AGG, OUT)),
            full_w((OUT, LBL)),
        ],
        out_specs=pl.BlockSpec((BB, LBL), lambda i: (i, 0)),
        out_shape=jax.ShapeDtypeStruct((B, LBL), jnp.float32),
        compiler_params=pltpu.CompilerParams(
            dimension_semantics=("arbitrary",),
        ),
    )(hop2.reshape(B, N1 * N2, F), hop1, target,
      wagg0, wagg1, wc0x, wc0a, wc1t, wc1a, W_cls)
    return out


# speedup vs baseline: 1.1975x; 1.0044x over previous
import functools

import jax
import jax.numpy as jnp
from jax.experimental import pallas as pl
from jax.experimental.pallas import tpu as pltpu

B, N1, N2, F = 10000, 8, 8, 128
AGG, OUT, LBL = 128, 128, 50
BB = 400


def _l2norm(x):
    s = jnp.sum(x * x, axis=-1, keepdims=True)
    return x * jax.lax.rsqrt(jnp.maximum(s, 1e-12))


def _fused_kernel(hop2_ref, hop1_ref, target_ref,
                  wagg0_ref, wagg1_ref,
                  wc0x_ref, wc0a_ref, wc1t_ref, wc1a_ref,
                  wcls_ref, out_ref):
    dot = functools.partial(jnp.dot, preferred_element_type=jnp.float32)
    wagg0 = wagg0_ref[...]

    x2 = hop2_ref[...].reshape(BB * N1 * N2, F)
    CH = 1024
    chunks = []
    for c in range(0, BB * N1 * N2, CH):
        pc = jax.nn.relu(dot(x2[c:c + CH], wagg0))
        pr = jnp.transpose(pc.reshape(CH // N2, N2, AGG), (1, 0, 2))
        chunks.append(jnp.sum(pr, axis=0))
    a_h2 = jnp.concatenate(chunks, axis=0)

    hop1 = hop1_ref[...].reshape(BB * N1, F)
    h1 = _l2norm(jax.nn.relu(dot(hop1, wc0x_ref[...]) + dot(a_h2, wc0a_ref[...])))

    a_h1 = jnp.sum(jnp.transpose(jax.nn.relu(dot(hop1, wagg0)).reshape(BB, N1, AGG), (1, 0, 2)), axis=0)

    t = _l2norm(jax.nn.relu(dot(target_ref[...], wc0x_ref[...]) + dot(a_h1, wc0a_ref[...])))

    a_l1 = jnp.sum(jnp.transpose(jax.nn.relu(dot(h1, wagg1_ref[...])).reshape(BB, N1, AGG), (1, 0, 2)), axis=0)

    full = _l2norm(dot(t, wc1t_ref[...]) + dot(a_l1, wc1a_ref[...]))
    full = _l2norm(full)

    out_ref[...] = jax.nn.relu(dot(full, wcls_ref[...]))


def kernel(hop2, hop1, target, W_agg0, W_agg1, W_comb0, W_comb1, W_cls):
    wagg0 = W_agg0 * (1.0 / N2)
    wagg1 = W_agg1 * (1.0 / N1)
    wc0x, wc0a = W_comb0[:F], W_comb0[F:]
    wc1t, wc1a = W_comb1[:OUT], W_comb1[OUT:]

    grid = (B // BB,)
    full_w = lambda shape: pl.BlockSpec(shape, lambda i: (0,) * len(shape))
    out = pl.pallas_call(
        _fused_kernel,
        grid=grid,
        in_specs=[
            pl.BlockSpec((BB, N1 * N2, F), lambda i: (i, 0, 0)),
            pl.BlockSpec((BB, N1, F), lambda i: (i, 0, 0)),
            pl.BlockSpec((BB, F), lambda i: (i, 0)),
            full_w((F, AGG)),
            full_w((OUT, AGG)),
            full_w((F, OUT)),
            full_w((AGG, OUT)),
            full_w((OUT, OUT)),
            full_w((AGG, OUT)),
            full_w((OUT, LBL)),
        ],
        out_specs=pl.BlockSpec((BB, LBL), lambda i: (i, 0)),
        out_shape=jax.ShapeDtypeStruct((B, LBL), jnp.float32),
        compiler_params=pltpu.CompilerParams(
            dimension_semantics=("arbitrary",),
        ),
    )(hop2.reshape(B, N1 * N2, F), hop1, target,
      wagg0, wagg1, wc0x, wc0a, wc1t, wc1a, W_cls)
    return out
